# baseline (speedup 1.0000x reference)
import functools

import jax
import jax.numpy as jnp
from jax import lax
from jax.experimental import pallas as pl
from jax.experimental.pallas import tpu as pltpu
from jax.experimental.pallas import tpu_sc as plsc

D = 64
NC, NS = 2, 16
NW = NC * NS
NB = 4


def _make_emb(batch, hist):
    rows_per_w = batch // NW
    n_iters = rows_per_w // NB

    mesh = plsc.VectorSubcoreMesh(core_axis_name="c", subcore_axis_name="s")

    @functools.partial(
        pl.kernel,
        mesh=mesh,
        out_type=jax.ShapeDtypeStruct((batch, hist, D), jnp.float32),
        compiler_params=pltpu.CompilerParams(
            use_tc_tiling_on_sc=False, skip_device_barrier=True),
        scratch_types=[
            pltpu.VMEM((2, NB, hist), jnp.int32),
            pltpu.VMEM((2, NB, hist, D), jnp.float32),
            pltpu.SemaphoreType.DMA,
            pltpu.SemaphoreType.DMA,
            pltpu.SemaphoreType.DMA,
            pltpu.SemaphoreType.DMA,
            pltpu.SemaphoreType.DMA,
            pltpu.SemaphoreType.DMA,
        ],
    )
    def emb(idx_hbm, table_hbm, out_hbm, idx_v, rows_v,
            gsem0, gsem1, osem0, osem1, isem0, isem1):
        gsem = (gsem0, gsem1)
        osem = (osem0, osem1)
        isem = (isem0, isem1)
        wid = lax.axis_index("s") * NC + lax.axis_index("c")
        row0 = wid * rows_per_w

        def fire_idx(b, g):
            pltpu.async_copy(
                idx_hbm.at[pl.ds(row0 + g * NB, NB)], idx_v.at[b], isem[b])

        def wait_idx(b, g):
            pltpu.make_async_copy(
                idx_hbm.at[pl.ds(row0 + g * NB, NB)], idx_v.at[b],
                isem[b]).wait()

        def fire_gathers(b, g):
            for j in range(NB):
                pltpu.async_copy(
                    table_hbm.at[idx_v.at[b, j]], rows_v.at[b, j], gsem[b])

        def wait_gathers(b):
            for j in range(NB):
                pltpu.make_async_copy(
                    table_hbm.at[idx_v.at[b, j]], rows_v.at[b, j],
                    gsem[b]).wait()

        def fire_out(b, g):
            pltpu.async_copy(
                rows_v.at[b], out_hbm.at[pl.ds(row0 + g * NB, NB)], osem[b])

        def wait_out(b, g):
            pltpu.make_async_copy(
                rows_v.at[b], out_hbm.at[pl.ds(row0 + g * NB, NB)],
                osem[b]).wait()

        fire_idx(0, 0)
        fire_idx(1, 1)

        @pl.loop(0, n_iters, step=2)
        def _(g0):
            for b in range(2):
                g = g0 + b

                @pl.when(g >= 2)
                def _():
                    wait_out(b, g - 2)

                wait_idx(b, g)
                fire_gathers(b, g)

                @pl.when(g >= 1)
                def _():
                    wait_gathers(1 - b)
                    fire_out(1 - b, g - 1)

                    @pl.when(g + 1 < n_iters)
                    def _():
                        fire_idx(1 - b, g + 1)

        last = n_iters - 1
        lb = last % 2
        wait_gathers(lb)
        fire_out(lb, last)
        wait_out(1 - lb, last - 1)
        wait_out(lb, last)

    return emb


HB = 128
BM = 2048


def _make_detrans(vocab):
    def body(in_ref, out_ref):
        x = in_ref[...]
        out_ref[...] = pltpu.einshape("d(re)->r(ed)", x, e=2)

    return pl.pallas_call(
        body,
        grid=((vocab + BM - 1) // BM,),
        in_specs=[pl.BlockSpec((D, BM), lambda i: (0, i))],
        out_specs=pl.BlockSpec((BM // 2, 2 * D), lambda i: (i, 0)),
        out_shape=jax.ShapeDtypeStruct((vocab // 2, 2 * D), jnp.float32),
    )


def _make_retile(batch, hist, n_halves, half):
    q = hist * D // 128
    groups = batch // n_halves // HB
    off = half * groups

    def body(*refs):
        in_ref, out_ref = refs[0], refs[-1]
        x = in_ref[...]
        x3 = x.reshape(HB, q, 128)
        y = jnp.transpose(x3, (1, 2, 0))
        out_ref[...] = y.reshape(hist, D, HB)

    in_specs = [pl.BlockSpec((HB * q, 128), lambda i: (i, 0))]
    kwargs = {}
    if half > 0:
        in_specs.append(pl.BlockSpec(memory_space=pl.ANY))
        kwargs["input_output_aliases"] = {1: 0}

    return pl.pallas_call(
        body,
        grid=(groups,),
        in_specs=in_specs,
        out_specs=pl.BlockSpec((hist, D, HB), lambda i: (0, 0, i + off)),
        out_shape=jax.ShapeDtypeStruct((hist, D, batch), jnp.float32),
        **kwargs,
    )


def kernel(tag_ids, table):
    batch, hist = tag_ids.shape
    idx = tag_ids.astype(jnp.int32)
    vocab = table.shape[0]
    table_lin = _make_detrans(vocab)(table.T).reshape(vocab, D)
    n_halves = 2
    hb = batch // n_halves
    out_t = None
    for half in range(n_halves):
        flat = _make_emb(hb, hist)(idx[half * hb:(half + 1) * hb], table_lin)
        flat2 = flat.reshape(hb * hist * D // 128, 128)
        if half == 0:
            out_t = _make_retile(batch, hist, n_halves, half)(flat2)
        else:
            out_t = _make_retile(batch, hist, n_halves, half)(flat2, out_t)
    return out_t.transpose(2, 0, 1)


# speedup vs baseline: 5.6409x; 5.6409x over previous
import functools

import jax
import jax.numpy as jnp
from jax import lax
from jax.experimental import pallas as pl
from jax.experimental.pallas import tpu as pltpu
from jax.experimental.pallas import tpu_sc as plsc

D = 64
NC, NS = 2, 16
NW = NC * NS
NB = 4


def _make_emb(batch, hist):
    rows_per_w = batch // NW
    n_iters = rows_per_w // NB

    mesh = plsc.VectorSubcoreMesh(core_axis_name="c", subcore_axis_name="s")

    @functools.partial(
        pl.kernel,
        mesh=mesh,
        out_type=jax.ShapeDtypeStruct((batch, hist, D), jnp.float32),
        compiler_params=pltpu.CompilerParams(
            use_tc_tiling_on_sc=False, skip_device_barrier=True),
        scratch_types=[
            pltpu.VMEM((2, NB, hist), jnp.int32),
            pltpu.VMEM((2, NB, hist, D), jnp.float32),
            pltpu.SemaphoreType.DMA,
            pltpu.SemaphoreType.DMA,
            pltpu.SemaphoreType.DMA,
            pltpu.SemaphoreType.DMA,
            pltpu.SemaphoreType.DMA,
            pltpu.SemaphoreType.DMA,
        ],
    )
    def emb(idx_hbm, table_hbm, out_hbm, idx_v, rows_v,
            gsem0, gsem1, osem0, osem1, isem0, isem1):
        gsem = (gsem0, gsem1)
        osem = (osem0, osem1)
        isem = (isem0, isem1)
        wid = lax.axis_index("s") * NC + lax.axis_index("c")
        row0 = wid * rows_per_w

        def fire_idx(b, g):
            pltpu.async_copy(
                idx_hbm.at[pl.ds(row0 + g * NB, NB)], idx_v.at[b], isem[b])

        def wait_idx(b, g):
            pltpu.make_async_copy(
                idx_hbm.at[pl.ds(row0 + g * NB, NB)], idx_v.at[b],
                isem[b]).wait()

        def fire_gathers(b, g):
            for j in range(NB):
                pltpu.async_copy(
                    table_hbm.at[idx_v.at[b, j]], rows_v.at[b, j], gsem[b])

        def wait_gathers(b):
            for j in range(NB):
                pltpu.make_async_copy(
                    table_hbm.at[idx_v.at[b, j]], rows_v.at[b, j],
                    gsem[b]).wait()

        def fire_out(b, g):
            pltpu.async_copy(
                rows_v.at[b], out_hbm.at[pl.ds(row0 + g * NB, NB)], osem[b])

        def wait_out(b, g):
            pltpu.make_async_copy(
                rows_v.at[b], out_hbm.at[pl.ds(row0 + g * NB, NB)],
                osem[b]).wait()

        fire_idx(0, 0)
        fire_idx(1, 1)

        @pl.loop(0, n_iters, step=2)
        def _(g0):
            for b in range(2):
                g = g0 + b

                @pl.when(g >= 2)
                def _():
                    wait_out(b, g - 2)

                wait_idx(b, g)
                fire_gathers(b, g)

                @pl.when(g >= 1)
                def _():
                    wait_gathers(1 - b)
                    fire_out(1 - b, g - 1)

                    @pl.when(g + 1 < n_iters)
                    def _():
                        fire_idx(1 - b, g + 1)

        last = n_iters - 1
        lb = last % 2
        wait_gathers(lb)
        fire_out(lb, last)
        wait_out(1 - lb, last - 1)
        wait_out(lb, last)

    return emb


HB = 128
BM = 2048


def _make_detrans(vocab):
    def body(in_ref, out_ref):
        y = in_ref[...].T
        y3 = y.reshape(BM // 2, 2, D)
        out_ref[:, 0:D] = y3[:, 0, :]
        out_ref[:, D:2 * D] = y3[:, 1, :]

    return pl.pallas_call(
        body,
        grid=((vocab + BM - 1) // BM,),
        in_specs=[pl.BlockSpec((D, BM), lambda i: (0, i))],
        out_specs=pl.BlockSpec((BM // 2, 2 * D), lambda i: (i, 0)),
        out_shape=jax.ShapeDtypeStruct((vocab // 2, 2 * D), jnp.float32),
    )


def _make_retile(batch, hist, n_halves, half):
    q = hist * D // 128
    groups = batch // n_halves // HB
    off = half * groups

    def body(*refs):
        in_ref, out_ref = refs[0], refs[-1]
        x = in_ref[...]
        x3 = x.reshape(HB, q, 128)
        y = jnp.transpose(x3, (1, 2, 0))
        out_ref[...] = y.reshape(hist, D, HB)

    in_specs = [pl.BlockSpec((HB * q, 128), lambda i: (i, 0))]
    kwargs = {}
    if half > 0:
        in_specs.append(pl.BlockSpec(memory_space=pl.ANY))
        kwargs["input_output_aliases"] = {1: 0}

    return pl.pallas_call(
        body,
        grid=(groups,),
        in_specs=in_specs,
        out_specs=pl.BlockSpec((hist, D, HB), lambda i: (0, 0, i + off)),
        out_shape=jax.ShapeDtypeStruct((hist, D, batch), jnp.float32),
        **kwargs,
    )


def kernel(tag_ids, table):
    batch, hist = tag_ids.shape
    idx = tag_ids.astype(jnp.int32)
    vocab = table.shape[0]
    table_lin = _make_detrans(vocab)(table.T).reshape(vocab, D)
    n_halves = 2
    hb = batch // n_halves
    out_t = None
    for half in range(n_halves):
        flat = _make_emb(hb, hist)(idx[half * hb:(half + 1) * hb], table_lin)
        flat2 = flat.reshape(hb * hist * D // 128, 128)
        if half == 0:
            out_t = _make_retile(batch, hist, n_halves, half)(flat2)
        else:
            out_t = _make_retile(batch, hist, n_halves, half)(flat2, out_t)
    return out_t.transpose(2, 0, 1)


# speedup vs baseline: 6.1857x; 1.0966x over previous
import functools

import jax
import jax.numpy as jnp
from jax import lax
from jax.experimental import pallas as pl
from jax.experimental.pallas import tpu as pltpu
from jax.experimental.pallas import tpu_sc as plsc

D = 64
NC, NS = 2, 16
NW = NC * NS
NB = 4


def _make_emb(batch, hist):
    rows_per_w = batch // NW
    n_iters = rows_per_w // NB

    mesh = plsc.VectorSubcoreMesh(core_axis_name="c", subcore_axis_name="s")

    @functools.partial(
        pl.kernel,
        mesh=mesh,
        out_type=jax.ShapeDtypeStruct((batch, hist, D), jnp.float32),
        compiler_params=pltpu.CompilerParams(
            use_tc_tiling_on_sc=False, skip_device_barrier=True),
        scratch_types=[
            pltpu.VMEM((2, NB, hist), jnp.int32),
            pltpu.VMEM((2, NB, hist, D), jnp.float32),
            pltpu.SemaphoreType.DMA,
            pltpu.SemaphoreType.DMA,
            pltpu.SemaphoreType.DMA,
            pltpu.SemaphoreType.DMA,
            pltpu.SemaphoreType.DMA,
            pltpu.SemaphoreType.DMA,
        ],
    )
    def emb(idx_hbm, table_hbm, out_hbm, idx_v, rows_v,
            gsem0, gsem1, osem0, osem1, isem0, isem1):
        gsem = (gsem0, gsem1)
        osem = (osem0, osem1)
        isem = (isem0, isem1)
        wid = lax.axis_index("s") * NC + lax.axis_index("c")
        row0 = wid * rows_per_w

        def fire_idx(b, g):
            pltpu.async_copy(
                idx_hbm.at[pl.ds(row0 + g * NB, NB)], idx_v.at[b], isem[b])

        def wait_idx(b, g):
            pltpu.make_async_copy(
                idx_hbm.at[pl.ds(row0 + g * NB, NB)], idx_v.at[b],
                isem[b]).wait()

        def fire_gathers(b, g):
            for j in range(NB):
                pltpu.async_copy(
                    table_hbm.at[idx_v.at[b, j]], rows_v.at[b, j], gsem[b])

        def wait_gathers(b):
            for j in range(NB):
                pltpu.make_async_copy(
                    table_hbm.at[idx_v.at[b, j]], rows_v.at[b, j],
                    gsem[b]).wait()

        def fire_out(b, g):
            pltpu.async_copy(
                rows_v.at[b], out_hbm.at[pl.ds(row0 + g * NB, NB)], osem[b])

        def wait_out(b, g):
            pltpu.make_async_copy(
                rows_v.at[b], out_hbm.at[pl.ds(row0 + g * NB, NB)],
                osem[b]).wait()

        fire_idx(0, 0)
        fire_idx(1, 1)

        @pl.loop(0, n_iters, step=2)
        def _(g0):
            for b in range(2):
                g = g0 + b

                @pl.when(g >= 2)
                def _():
                    wait_out(b, g - 2)

                wait_idx(b, g)
                fire_gathers(b, g)

                @pl.when(g >= 1)
                def _():
                    wait_gathers(1 - b)
                    fire_out(1 - b, g - 1)

                    @pl.when(g + 1 < n_iters)
                    def _():
                        fire_idx(1 - b, g + 1)

        last = n_iters - 1
        lb = last % 2
        wait_gathers(lb)
        fire_out(lb, last)
        wait_out(1 - lb, last - 1)
        wait_out(lb, last)

    return emb


HB = 128
BM = 8192


def _make_detrans(vocab):
    def body(in_ref, out_ref):
        y = in_ref[...].T
        y3 = y.reshape(BM // 2, 2, D)
        out_ref[:, 0:D] = y3[:, 0, :]
        out_ref[:, D:2 * D] = y3[:, 1, :]

    return pl.pallas_call(
        body,
        grid=((vocab + BM - 1) // BM,),
        in_specs=[pl.BlockSpec((D, BM), lambda i: (0, i))],
        out_specs=pl.BlockSpec((BM // 2, 2 * D), lambda i: (i, 0)),
        out_shape=jax.ShapeDtypeStruct((vocab // 2, 2 * D), jnp.float32),
    )


def _make_retile(batch, hist, n_halves, half):
    q = hist * D // 128
    groups = batch // n_halves // HB
    off = half * groups

    def body(*refs):
        in_ref, out_ref = refs[0], refs[-1]
        x = in_ref[...]
        x3 = x.reshape(HB, q, 128)
        y = jnp.transpose(x3, (1, 2, 0))
        out_ref[...] = y.reshape(hist, D, HB)

    in_specs = [pl.BlockSpec((HB * q, 128), lambda i: (i, 0))]
    kwargs = {}
    if half > 0:
        in_specs.append(pl.BlockSpec(memory_space=pl.ANY))
        kwargs["input_output_aliases"] = {1: 0}

    return pl.pallas_call(
        body,
        grid=(groups,),
        in_specs=in_specs,
        out_specs=pl.BlockSpec((hist, D, HB), lambda i: (0, 0, i + off)),
        out_shape=jax.ShapeDtypeStruct((hist, D, batch), jnp.float32),
        **kwargs,
    )


def kernel(tag_ids, table):
    batch, hist = tag_ids.shape
    idx = tag_ids.astype(jnp.int32)
    vocab = table.shape[0]
    table_lin = _make_detrans(vocab)(table.T).reshape(vocab, D)
    n_halves = 2
    hb = batch // n_halves
    out_t = None
    for half in range(n_halves):
        flat = _make_emb(hb, hist)(idx[half * hb:(half + 1) * hb], table_lin)
        flat2 = flat.reshape(hb * hist * D // 128, 128)
        if half == 0:
            out_t = _make_retile(batch, hist, n_halves, half)(flat2)
        else:
            out_t = _make_retile(batch, hist, n_halves, half)(flat2, out_t)
    return out_t.transpose(2, 0, 1)
